# baseline (speedup 1.0000x reference)
import functools
import math

import numpy as np
import jax
import jax.numpy as jnp
from jax import lax
from jax.experimental import pallas as pl
from jax.experimental.pallas import tpu as pltpu
from jax.experimental.pallas import tpu_sc as plsc

B, CIN, H, W = 4, 64, 256, 256
CF, HF, WF = 128, 64, 64
S = 128
NTILES = 32
TPB = NTILES // B
ROWS = H // TPB
NL = 16
GRP = W // NL


def _interp_matrix(out_size: int, in_size: int) -> np.ndarray:
    pos = np.linspace(0.0, in_size - 1.0, out_size)
    lo = np.floor(pos).astype(np.int64)
    hi = np.minimum(lo + 1, in_size - 1)
    w = (pos - lo).astype(np.float32)
    m = np.zeros((out_size, in_size), np.float32)
    m[np.arange(out_size), lo] += 1.0 - w
    m[np.arange(out_size), hi] += w
    return m

_R_NP = _interp_matrix(H, HF)


def _chansum_body(x_ref, f_ref, r_ref, rt_ref, o_ref, fo_ref):
    o_ref[...] = jnp.sum(x_ref[...], axis=1)
    fs = jnp.sum(f_ref[0], axis=-1)
    rows = jnp.dot(r_ref[...], fs, preferred_element_type=jnp.float32)
    fo_ref[0] = jnp.dot(rows, rt_ref[...], preferred_element_type=jnp.float32)


def _chansum(x, f, r, rt):
    return pl.pallas_call(
        _chansum_body,
        grid=(B,),
        in_specs=[
            pl.BlockSpec((1, CIN, H, W), lambda i: (i, 0, 0, 0)),
            pl.BlockSpec((1, HF, WF, CF), lambda i: (i, 0, 0, 0)),
            pl.BlockSpec((H, HF), lambda i: (0, 0)),
            pl.BlockSpec((HF, W), lambda i: (0, 0)),
        ],
        out_specs=[
            pl.BlockSpec((1, H, W), lambda i: (i, 0, 0)),
            pl.BlockSpec((1, H, W), lambda i: (i, 0, 0)),
        ],
        out_shape=[
            jax.ShapeDtypeStruct((B, H, W), jnp.float32),
            jax.ShapeDtypeStruct((B, H, W), jnp.float32),
        ],
    )(x, f, r, rt)


def _seg_body(vals_hbm, fup_hbm, sp_hbm, out_hbm, v_v, f_v, l_v, a1, a2, ac, part):
    wid = lax.axis_index("s") * 2 + lax.axis_index("c")
    b = wid // TPB
    r0 = (wid % TPB) * ROWS
    pltpu.sync_copy(vals_hbm.at[b, pl.ds(r0, ROWS), :], v_v)
    pltpu.sync_copy(fup_hbm.at[b, pl.ds(r0, ROWS), :], f_v)
    pltpu.sync_copy(sp_hbm.at[b, pl.ds(r0, ROWS), :], l_v)

    zeros = jnp.zeros((NL,), jnp.float32)

    def zero_body(j, carry):
        a1[pl.ds(j * NL, NL)] = zeros
        a2[pl.ds(j * NL, NL)] = zeros
        ac[pl.ds(j * NL, NL)] = zeros
        return carry

    lax.fori_loop(0, (S * NL) // NL, zero_body, 0)

    lane_off = lax.iota(jnp.int32, NL) * S
    ones = jnp.ones((NL,), jnp.float32)

    @plsc.parallel_loop(0, ROWS, unroll=4)
    def _scatter_rows(r):
        for g in range(GRP):
            idx = l_v[r, pl.ds(g * NL, NL)] + lane_off
            plsc.addupdate_scatter(a1, [idx], v_v[r, pl.ds(g * NL, NL)])
            plsc.addupdate_scatter(a2, [idx], f_v[r, pl.ds(g * NL, NL)])
            plsc.addupdate_scatter(ac, [idx], ones)

    for j in range(S // NL):
        s1 = a1[pl.ds(j * NL, NL)]
        s2 = a2[pl.ds(j * NL, NL)]
        sc = ac[pl.ds(j * NL, NL)]
        for l in range(1, NL):
            off = l * S + j * NL
            s1 = s1 + a1[pl.ds(off, NL)]
            s2 = s2 + a2[pl.ds(off, NL)]
            sc = sc + ac[pl.ds(off, NL)]
        part[pl.ds(j * NL, NL)] = s1
        part[pl.ds(S + j * NL, NL)] = s2
        part[pl.ds(2 * S + j * NL, NL)] = sc

    pltpu.sync_copy(part, out_hbm.at[wid])


@functools.cache
def _seg_reduce():
    return pl.kernel(
        _seg_body,
        out_type=jax.ShapeDtypeStruct((NTILES, 3 * S), jnp.float32),
        mesh=plsc.VectorSubcoreMesh(core_axis_name="c", subcore_axis_name="s"),
        compiler_params=pltpu.CompilerParams(needs_layout_passes=False),
        scratch_types=[
            pltpu.VMEM((ROWS, W), jnp.float32),
            pltpu.VMEM((ROWS, W), jnp.float32),
            pltpu.VMEM((ROWS, W), jnp.int32),
            pltpu.VMEM((S * NL,), jnp.float32),
            pltpu.VMEM((S * NL,), jnp.float32),
            pltpu.VMEM((S * NL,), jnp.float32),
            pltpu.VMEM((3 * S,), jnp.float32),
        ],
    )


def _final_body(p_ref, o_ref):
    rows = p_ref[...]
    q = jnp.concatenate(
        [jnp.sum(rows[b * TPB:(b + 1) * TPB], axis=0, keepdims=True)
         for b in range(B)],
        axis=0,
    )
    s1 = q[:, 0:S]
    s2 = q[:, S:2 * S]
    cnt = q[:, 2 * S:3 * S]
    m1 = s1 / (cnt * float(CIN))
    m2 = s2 / (cnt * float(CF))
    d1 = jnp.abs(m1[:, :, None] - m1[:, None, :])
    d2 = jnp.abs(m2[:, :, None] - m2[:, None, :])
    tot = jnp.sum(jnp.abs(d1 - d2)) * (1.0 / float(B * S * S))
    o_ref[...] = jnp.broadcast_to(tot, (1, 1))


def _final(partials):
    return pl.pallas_call(
        _final_body,
        in_specs=[pl.BlockSpec((NTILES, 3 * S), lambda: (0, 0))],
        out_specs=pl.BlockSpec((1, 1), lambda: (0, 0)),
        out_shape=jax.ShapeDtypeStruct((1, 1), jnp.float32),
    )(partials)


def kernel(input, feature, sp, num):
    del num
    r = jnp.asarray(_R_NP)
    vals, fup = _chansum(input, jnp.transpose(feature, (0, 2, 3, 1)), r, r.T)
    partials = _seg_reduce()(vals, fup, sp.reshape(B, H, W))
    out = _final(partials)
    return out[0, 0]


# speedup vs baseline: 1.0014x; 1.0014x over previous
import functools
import math

import numpy as np
import jax
import jax.numpy as jnp
from jax import lax
from jax.experimental import pallas as pl
from jax.experimental.pallas import tpu as pltpu
from jax.experimental.pallas import tpu_sc as plsc

B, CIN, H, W = 4, 64, 256, 256
CF, HF, WF = 128, 64, 64
S = 128
NTILES = 32
TPB = NTILES // B
ROWS = H // TPB
NL = 16
GRP = W // NL


def _interp_matrix(out_size: int, in_size: int) -> np.ndarray:
    pos = np.linspace(0.0, in_size - 1.0, out_size)
    lo = np.floor(pos).astype(np.int64)
    hi = np.minimum(lo + 1, in_size - 1)
    w = (pos - lo).astype(np.float32)
    m = np.zeros((out_size, in_size), np.float32)
    m[np.arange(out_size), lo] += 1.0 - w
    m[np.arange(out_size), hi] += w
    return m

_R_NP = _interp_matrix(H, HF)


def _chansum_body(x_ref, f_ref, r_ref, rt_ref, o_ref, fo_ref):
    o_ref[...] = jnp.sum(x_ref[...], axis=1)
    fs = jnp.sum(f_ref[0], axis=-1)
    rows = jnp.dot(r_ref[...], fs, preferred_element_type=jnp.float32)
    fo_ref[0] = jnp.dot(rows, rt_ref[...], preferred_element_type=jnp.float32)


def _chansum(x, f, r, rt):
    return pl.pallas_call(
        _chansum_body,
        grid=(B,),
        in_specs=[
            pl.BlockSpec((1, CIN, H, W), lambda i: (i, 0, 0, 0)),
            pl.BlockSpec((1, HF, WF, CF), lambda i: (i, 0, 0, 0)),
            pl.BlockSpec((H, HF), lambda i: (0, 0)),
            pl.BlockSpec((HF, W), lambda i: (0, 0)),
        ],
        out_specs=[
            pl.BlockSpec((1, H, W), lambda i: (i, 0, 0)),
            pl.BlockSpec((1, H, W), lambda i: (i, 0, 0)),
        ],
        out_shape=[
            jax.ShapeDtypeStruct((B, H, W), jnp.float32),
            jax.ShapeDtypeStruct((B, H, W), jnp.float32),
        ],
    )(x, f, r, rt)


def _seg_body(vals_hbm, fup_hbm, sp_hbm, out_hbm, v_v, f_v, l_v, a1, a2, ac, part):
    wid = lax.axis_index("s") * 2 + lax.axis_index("c")
    b = wid // TPB
    r0 = (wid % TPB) * ROWS
    pltpu.sync_copy(vals_hbm.at[b, pl.ds(r0, ROWS), :], v_v)
    pltpu.sync_copy(fup_hbm.at[b, pl.ds(r0, ROWS), :], f_v)
    pltpu.sync_copy(sp_hbm.at[b, pl.ds(r0, ROWS), :], l_v)

    zeros = jnp.zeros((NL,), jnp.float32)

    def zero_body(j, carry):
        a1[pl.ds(j * NL, NL)] = zeros
        a2[pl.ds(j * NL, NL)] = zeros
        ac[pl.ds(j * NL, NL)] = zeros
        return carry

    lax.fori_loop(0, (S * NL) // NL, zero_body, 0)

    lane_off = lax.iota(jnp.int32, NL) * S
    ones = jnp.ones((NL,), jnp.float32)

    @plsc.parallel_loop(0, ROWS, unroll=2)
    def _scatter_rows(r):
        for g in range(GRP):
            idx = l_v[r, pl.ds(g * NL, NL)] + lane_off
            plsc.addupdate_scatter(a1, [idx], v_v[r, pl.ds(g * NL, NL)])
            plsc.addupdate_scatter(a2, [idx], f_v[r, pl.ds(g * NL, NL)])
            plsc.addupdate_scatter(ac, [idx], ones)

    for j in range(S // NL):
        s1 = a1[pl.ds(j * NL, NL)]
        s2 = a2[pl.ds(j * NL, NL)]
        sc = ac[pl.ds(j * NL, NL)]
        for l in range(1, NL):
            off = l * S + j * NL
            s1 = s1 + a1[pl.ds(off, NL)]
            s2 = s2 + a2[pl.ds(off, NL)]
            sc = sc + ac[pl.ds(off, NL)]
        part[pl.ds(j * NL, NL)] = s1
        part[pl.ds(S + j * NL, NL)] = s2
        part[pl.ds(2 * S + j * NL, NL)] = sc

    pltpu.sync_copy(part, out_hbm.at[wid])


@functools.cache
def _seg_reduce():
    return pl.kernel(
        _seg_body,
        out_type=jax.ShapeDtypeStruct((NTILES, 3 * S), jnp.float32),
        mesh=plsc.VectorSubcoreMesh(core_axis_name="c", subcore_axis_name="s"),
        compiler_params=pltpu.CompilerParams(needs_layout_passes=False),
        scratch_types=[
            pltpu.VMEM((ROWS, W), jnp.float32),
            pltpu.VMEM((ROWS, W), jnp.float32),
            pltpu.VMEM((ROWS, W), jnp.int32),
            pltpu.VMEM((S * NL,), jnp.float32),
            pltpu.VMEM((S * NL,), jnp.float32),
            pltpu.VMEM((S * NL,), jnp.float32),
            pltpu.VMEM((3 * S,), jnp.float32),
        ],
    )


def _final_body(p_ref, o_ref):
    rows = p_ref[...]
    q = jnp.concatenate(
        [jnp.sum(rows[b * TPB:(b + 1) * TPB], axis=0, keepdims=True)
         for b in range(B)],
        axis=0,
    )
    s1 = q[:, 0:S]
    s2 = q[:, S:2 * S]
    cnt = q[:, 2 * S:3 * S]
    m1 = s1 / (cnt * float(CIN))
    m2 = s2 / (cnt * float(CF))
    d1 = jnp.abs(m1[:, :, None] - m1[:, None, :])
    d2 = jnp.abs(m2[:, :, None] - m2[:, None, :])
    tot = jnp.sum(jnp.abs(d1 - d2)) * (1.0 / float(B * S * S))
    o_ref[...] = jnp.broadcast_to(tot, (1, 1))


def _final(partials):
    return pl.pallas_call(
        _final_body,
        in_specs=[pl.BlockSpec((NTILES, 3 * S), lambda: (0, 0))],
        out_specs=pl.BlockSpec((1, 1), lambda: (0, 0)),
        out_shape=jax.ShapeDtypeStruct((1, 1), jnp.float32),
    )(partials)


def kernel(input, feature, sp, num):
    del num
    r = jnp.asarray(_R_NP)
    vals, fup = _chansum(input, jnp.transpose(feature, (0, 2, 3, 1)), r, r.T)
    partials = _seg_reduce()(vals, fup, sp.reshape(B, H, W))
    out = _final(partials)
    return out[0, 0]


# speedup vs baseline: 1.0022x; 1.0008x over previous
import functools

import numpy as np
import jax
import jax.numpy as jnp
from jax import lax
from jax.experimental import pallas as pl
from jax.experimental.pallas import tpu as pltpu
from jax.experimental.pallas import tpu_sc as plsc

B, CIN, H, W = 4, 64, 256, 256
CF, HF, WF = 128, 64, 64
S = 128
NTILES = 32
TPB = NTILES // B
ROWS = H // TPB
NL = 16
GRP = W // NL


def _interp_matrix(out_size: int, in_size: int) -> np.ndarray:
    pos = np.linspace(0.0, in_size - 1.0, out_size)
    lo = np.floor(pos).astype(np.int64)
    hi = np.minimum(lo + 1, in_size - 1)
    w = (pos - lo).astype(np.float32)
    m = np.zeros((out_size, in_size), np.float32)
    m[np.arange(out_size), lo] += 1.0 - w
    m[np.arange(out_size), hi] += w
    return m

_R_NP = _interp_matrix(H, HF)


def _chansum_body(x_ref, f_ref, r_ref, rt_ref, o_ref, fo_ref):
    o_ref[...] = jnp.sum(x_ref[...], axis=1)
    fs = jnp.sum(f_ref[0], axis=-1)
    rows = jnp.dot(r_ref[...], fs, preferred_element_type=jnp.float32)
    fo_ref[0] = jnp.dot(rows, rt_ref[...], preferred_element_type=jnp.float32)


def _chansum(x, f, r, rt):
    return pl.pallas_call(
        _chansum_body,
        grid=(B,),
        in_specs=[
            pl.BlockSpec((1, CIN, H, W), lambda i: (i, 0, 0, 0)),
            pl.BlockSpec((1, HF, WF, CF), lambda i: (i, 0, 0, 0)),
            pl.BlockSpec((H, HF), lambda i: (0, 0)),
            pl.BlockSpec((HF, W), lambda i: (0, 0)),
        ],
        out_specs=[
            pl.BlockSpec((1, H, W), lambda i: (i, 0, 0)),
            pl.BlockSpec((1, H, W), lambda i: (i, 0, 0)),
        ],
        out_shape=[
            jax.ShapeDtypeStruct((B, H, W), jnp.float32),
            jax.ShapeDtypeStruct((B, H, W), jnp.float32),
        ],
    )(x, f, r, rt)


def _seg_body(vals_hbm, fup_hbm, sp_hbm, out_hbm, v_v, f_v, l_v, a1, a2, ac, part):
    wid = lax.axis_index("s") * 2 + lax.axis_index("c")
    b = wid // TPB
    r0 = (wid % TPB) * ROWS
    pltpu.sync_copy(vals_hbm.at[b, pl.ds(r0, ROWS), :], v_v)
    pltpu.sync_copy(fup_hbm.at[b, pl.ds(r0, ROWS), :], f_v)
    pltpu.sync_copy(sp_hbm.at[b, pl.ds(r0, ROWS), :], l_v)

    zeros = jnp.zeros((NL,), jnp.float32)

    def zero_body(j, carry):
        a1[pl.ds(j * NL, NL)] = zeros
        a2[pl.ds(j * NL, NL)] = zeros
        ac[pl.ds(j * NL, NL)] = zeros
        return carry

    lax.fori_loop(0, (S * NL) // NL, zero_body, 0)

    lane_off = lax.iota(jnp.int32, NL) * S
    ones = jnp.ones((NL,), jnp.float32)

    @plsc.parallel_loop(0, ROWS, unroll=2)
    def _scatter_rows(r):
        for g in range(GRP):
            idx = l_v[r, pl.ds(g * NL, NL)] + lane_off
            plsc.addupdate_scatter(a1, [idx], v_v[r, pl.ds(g * NL, NL)])
            plsc.addupdate_scatter(a2, [idx], f_v[r, pl.ds(g * NL, NL)])
            plsc.addupdate_scatter(ac, [idx], ones)

    for j in range(S // NL):
        s1 = a1[pl.ds(j * NL, NL)]
        s2 = a2[pl.ds(j * NL, NL)]
        sc = ac[pl.ds(j * NL, NL)]
        for l in range(1, NL):
            off = l * S + j * NL
            s1 = s1 + a1[pl.ds(off, NL)]
            s2 = s2 + a2[pl.ds(off, NL)]
            sc = sc + ac[pl.ds(off, NL)]
        part[pl.ds(j * NL, NL)] = s1
        part[pl.ds(S + j * NL, NL)] = s2
        part[pl.ds(2 * S + j * NL, NL)] = sc

    pltpu.sync_copy(part, out_hbm.at[wid])


@functools.cache
def _seg_reduce():
    return pl.kernel(
        _seg_body,
        out_type=jax.ShapeDtypeStruct((NTILES, 3 * S), jnp.float32),
        mesh=plsc.VectorSubcoreMesh(core_axis_name="c", subcore_axis_name="s"),
        compiler_params=pltpu.CompilerParams(needs_layout_passes=False),
        scratch_types=[
            pltpu.VMEM((ROWS, W), jnp.float32),
            pltpu.VMEM((ROWS, W), jnp.float32),
            pltpu.VMEM((ROWS, W), jnp.int32),
            pltpu.VMEM((S * NL,), jnp.float32),
            pltpu.VMEM((S * NL,), jnp.float32),
            pltpu.VMEM((S * NL,), jnp.float32),
            pltpu.VMEM((3 * S,), jnp.float32),
        ],
    )


def _final_body(p_ref, o_ref):
    rows = p_ref[...]
    q = jnp.concatenate(
        [jnp.sum(rows[b * TPB:(b + 1) * TPB], axis=0, keepdims=True)
         for b in range(B)],
        axis=0,
    )
    s1 = q[:, 0:S]
    s2 = q[:, S:2 * S]
    cnt = q[:, 2 * S:3 * S]
    m1 = s1 / (cnt * float(CIN))
    m2 = s2 / (cnt * float(CF))
    d1 = jnp.abs(m1[:, :, None] - m1[:, None, :])
    d2 = jnp.abs(m2[:, :, None] - m2[:, None, :])
    tot = jnp.sum(jnp.abs(d1 - d2)) * (1.0 / float(B * S * S))
    o_ref[...] = jnp.broadcast_to(tot, (1, 1))


def _final(partials):
    return pl.pallas_call(
        _final_body,
        in_specs=[pl.BlockSpec((NTILES, 3 * S), lambda: (0, 0))],
        out_specs=pl.BlockSpec((1, 1), lambda: (0, 0)),
        out_shape=jax.ShapeDtypeStruct((1, 1), jnp.float32),
    )(partials)


def kernel(input, feature, sp, num):
    del num
    r = jnp.asarray(_R_NP)
    vals, fup = _chansum(input, jnp.transpose(feature, (0, 2, 3, 1)), r, r.T)
    partials = _seg_reduce()(vals, fup, sp.reshape(B, H, W))
    out = _final(partials)
    return out[0, 0]


# speedup vs baseline: 1.0030x; 1.0008x over previous
import functools

import numpy as np
import jax
import jax.numpy as jnp
from jax import lax
from jax.experimental import pallas as pl
from jax.experimental.pallas import tpu as pltpu
from jax.experimental.pallas import tpu_sc as plsc

B, CIN, H, W = 4, 64, 256, 256
CF, HF, WF = 128, 64, 64
S = 128
NTILES = 32
TPB = NTILES // B
ROWS = H // TPB
NL = 16
GRP = W // NL


def _interp_matrix(out_size: int, in_size: int) -> np.ndarray:
    pos = np.linspace(0.0, in_size - 1.0, out_size)
    lo = np.floor(pos).astype(np.int64)
    hi = np.minimum(lo + 1, in_size - 1)
    w = (pos - lo).astype(np.float32)
    m = np.zeros((out_size, in_size), np.float32)
    m[np.arange(out_size), lo] += 1.0 - w
    m[np.arange(out_size), hi] += w
    return m

_R_NP = _interp_matrix(H, HF)


def _chansum_body(x_ref, f_ref, o_ref, fo_ref):
    o_ref[...] = jnp.sum(x_ref[...], axis=1)
    fs = jnp.sum(f_ref[0], axis=-1)
    o_pos = jax.lax.broadcasted_iota(jnp.int32, (H, HF), 0).astype(jnp.float32)
    h_pos = jax.lax.broadcasted_iota(jnp.int32, (H, HF), 1).astype(jnp.float32)
    r = jnp.maximum(0.0, 1.0 - jnp.abs(o_pos * (float(HF - 1) / (H - 1)) - h_pos))
    rows = jnp.dot(r, fs, preferred_element_type=jnp.float32)
    fo_ref[0] = jax.lax.dot_general(
        rows, r, (((1,), (1,)), ((), ())),
        preferred_element_type=jnp.float32,
    )


def _chansum(x, f):
    return pl.pallas_call(
        _chansum_body,
        grid=(B,),
        in_specs=[
            pl.BlockSpec((1, CIN, H, W), lambda i: (i, 0, 0, 0)),
            pl.BlockSpec((1, HF, WF, CF), lambda i: (i, 0, 0, 0)),
        ],
        out_specs=[
            pl.BlockSpec((1, H, W), lambda i: (i, 0, 0)),
            pl.BlockSpec((1, H, W), lambda i: (i, 0, 0)),
        ],
        out_shape=[
            jax.ShapeDtypeStruct((B, H, W), jnp.float32),
            jax.ShapeDtypeStruct((B, H, W), jnp.float32),
        ],
    )(x, f)


def _seg_body(vals_hbm, fup_hbm, sp_hbm, out_hbm, v_v, f_v, l_v, a1, a2, ac, part):
    wid = lax.axis_index("s") * 2 + lax.axis_index("c")
    b = wid // TPB
    r0 = (wid % TPB) * ROWS
    pltpu.sync_copy(vals_hbm.at[b, pl.ds(r0, ROWS), :], v_v)
    pltpu.sync_copy(fup_hbm.at[b, pl.ds(r0, ROWS), :], f_v)
    pltpu.sync_copy(sp_hbm.at[b, pl.ds(r0, ROWS), :], l_v)

    zeros = jnp.zeros((NL,), jnp.float32)

    def zero_body(j, carry):
        a1[pl.ds(j * NL, NL)] = zeros
        a2[pl.ds(j * NL, NL)] = zeros
        ac[pl.ds(j * NL, NL)] = zeros
        return carry

    lax.fori_loop(0, (S * NL) // NL, zero_body, 0)

    lane_off = lax.iota(jnp.int32, NL) * S
    ones = jnp.ones((NL,), jnp.float32)

    @plsc.parallel_loop(0, ROWS, unroll=2)
    def _scatter_rows(r):
        for g in range(GRP):
            idx = l_v[r, pl.ds(g * NL, NL)] + lane_off
            plsc.addupdate_scatter(a1, [idx], v_v[r, pl.ds(g * NL, NL)])
            plsc.addupdate_scatter(a2, [idx], f_v[r, pl.ds(g * NL, NL)])
            plsc.addupdate_scatter(ac, [idx], ones)

    for j in range(S // NL):
        s1 = a1[pl.ds(j * NL, NL)]
        s2 = a2[pl.ds(j * NL, NL)]
        sc = ac[pl.ds(j * NL, NL)]
        for l in range(1, NL):
            off = l * S + j * NL
            s1 = s1 + a1[pl.ds(off, NL)]
            s2 = s2 + a2[pl.ds(off, NL)]
            sc = sc + ac[pl.ds(off, NL)]
        part[pl.ds(j * NL, NL)] = s1
        part[pl.ds(S + j * NL, NL)] = s2
        part[pl.ds(2 * S + j * NL, NL)] = sc

    pltpu.sync_copy(part, out_hbm.at[wid])


@functools.cache
def _seg_reduce():
    return pl.kernel(
        _seg_body,
        out_type=jax.ShapeDtypeStruct((NTILES, 3 * S), jnp.float32),
        mesh=plsc.VectorSubcoreMesh(core_axis_name="c", subcore_axis_name="s"),
        compiler_params=pltpu.CompilerParams(needs_layout_passes=False),
        scratch_types=[
            pltpu.VMEM((ROWS, W), jnp.float32),
            pltpu.VMEM((ROWS, W), jnp.float32),
            pltpu.VMEM((ROWS, W), jnp.int32),
            pltpu.VMEM((S * NL,), jnp.float32),
            pltpu.VMEM((S * NL,), jnp.float32),
            pltpu.VMEM((S * NL,), jnp.float32),
            pltpu.VMEM((3 * S,), jnp.float32),
        ],
    )


def _final_body(p_ref, o_ref):
    rows = p_ref[...]
    q = jnp.concatenate(
        [jnp.sum(rows[b * TPB:(b + 1) * TPB], axis=0, keepdims=True)
         for b in range(B)],
        axis=0,
    )
    s1 = q[:, 0:S]
    s2 = q[:, S:2 * S]
    cnt = q[:, 2 * S:3 * S]
    m1 = s1 / (cnt * float(CIN))
    m2 = s2 / (cnt * float(CF))
    d1 = jnp.abs(m1[:, :, None] - m1[:, None, :])
    d2 = jnp.abs(m2[:, :, None] - m2[:, None, :])
    tot = jnp.sum(jnp.abs(d1 - d2)) * (1.0 / float(B * S * S))
    o_ref[...] = jnp.broadcast_to(tot, (1, 1))


def _final(partials):
    return pl.pallas_call(
        _final_body,
        in_specs=[pl.BlockSpec((NTILES, 3 * S), lambda: (0, 0))],
        out_specs=pl.BlockSpec((1, 1), lambda: (0, 0)),
        out_shape=jax.ShapeDtypeStruct((1, 1), jnp.float32),
    )(partials)


def kernel(input, feature, sp, num):
    del num
    vals, fup = _chansum(input, jnp.transpose(feature, (0, 2, 3, 1)))
    partials = _seg_reduce()(vals, fup, sp.reshape(B, H, W))
    out = _final(partials)
    return out[0, 0]
